# Initial kernel scaffold; baseline (speedup 1.0000x reference)
import jax
import jax.numpy as jnp
from jax.experimental import pallas as pl


def kernel(prefix, table):
    raise NotImplementedError("write your pallas kernel here")


# speedup vs baseline: 1.6412x; 1.6412x over previous
import functools

import jax
import jax.numpy as jnp
from jax import lax
from jax.experimental import pallas as pl
from jax.experimental.pallas import tpu as pltpu
from jax.experimental.pallas import tpu_sc as plsc

PREFIX_LENGTH = 128
NUM_LAYERS = 24
HIDDEN_SIZE = 1024
BATCH = 16
EMBED_DIM = NUM_LAYERS * HIDDEN_SIZE
B = BATCH * PREFIX_LENGTH

NC, NS = 2, 16
NW = NC * NS
BPW = B // NW
CH = 4
NCHUNK = BPW // CH

_mesh = plsc.VectorSubcoreMesh(core_axis_name="c", subcore_axis_name="s")


@functools.partial(
    pl.kernel,
    mesh=_mesh,
    out_type=jax.ShapeDtypeStruct((B, EMBED_DIM), jnp.float32),
    scratch_types=[
        pltpu.VMEM((NCHUNK, CH), jnp.int32),
        pltpu.VMEM((CH, EMBED_DIM), jnp.float32),
        pltpu.SemaphoreType.DMA,
    ],
)
def _gather_kernel(idx_hbm, table_hbm, out_hbm, idx_v, rows_v, sem):
    wid = lax.axis_index("s") * NC + lax.axis_index("c")
    pltpu.sync_copy(idx_hbm.at[wid], idx_v)

    def body(c, carry):
        pltpu.async_copy(table_hbm.at[idx_v.at[c]], rows_v, sem).wait()
        pltpu.sync_copy(rows_v, out_hbm.at[pl.ds(wid * BPW + c * CH, CH)])
        return carry

    lax.fori_loop(0, NCHUNK, body, 0)


def kernel(prefix, table):
    idx = prefix.astype(jnp.int32).reshape(NW, NCHUNK, CH)
    out = _gather_kernel(idx, table)
    return out.reshape(BATCH, PREFIX_LENGTH, EMBED_DIM)


# speedup vs baseline: 1.7292x; 1.0536x over previous
import functools

import jax
import jax.numpy as jnp
from jax import lax
from jax.experimental import pallas as pl
from jax.experimental.pallas import tpu as pltpu
from jax.experimental.pallas import tpu_sc as plsc

PREFIX_LENGTH = 128
NUM_LAYERS = 24
HIDDEN_SIZE = 1024
BATCH = 16
EMBED_DIM = NUM_LAYERS * HIDDEN_SIZE
B = BATCH * PREFIX_LENGTH

NC, NS = 2, 16
NW = NC * NS
BPW = B // NW
CH = 2
NCHUNK = BPW // CH
NBUF = 2
NGROUP = NCHUNK // NBUF

_mesh = plsc.VectorSubcoreMesh(core_axis_name="c", subcore_axis_name="s")


@functools.partial(
    pl.kernel,
    mesh=_mesh,
    out_type=jax.ShapeDtypeStruct((B, EMBED_DIM), jnp.float32),
    scratch_types=[
        pltpu.VMEM((NCHUNK, CH), jnp.int32),
        pltpu.VMEM((CH, EMBED_DIM), jnp.float32),
        pltpu.VMEM((CH, EMBED_DIM), jnp.float32),
        pltpu.SemaphoreType.DMA,
        pltpu.SemaphoreType.DMA,
        pltpu.SemaphoreType.DMA,
        pltpu.SemaphoreType.DMA,
    ],
)
def _gather_kernel(idx_hbm, table_hbm, out_hbm, idx_v,
                   rows0, rows1, gsem0, gsem1, ssem0, ssem1):
    wid = lax.axis_index("s") * NC + lax.axis_index("c")
    base = wid * BPW
    rows = (rows0, rows1)
    gsem = (gsem0, gsem1)
    ssem = (ssem0, ssem1)
    pltpu.sync_copy(idx_hbm.at[wid], idx_v)

    def body(g, carry):
        for b in range(NBUF):
            c = NBUF * g + b

            @pl.when(g > 0)
            def _wait_prev_store():
                pltpu.make_async_copy(
                    rows[b],
                    out_hbm.at[pl.ds(base + (c - NBUF) * CH, CH)],
                    ssem[b],
                ).wait()

            pltpu.async_copy(
                table_hbm.at[idx_v.at[c]], rows[b], gsem[b]
            ).wait()
            pltpu.async_copy(
                rows[b], out_hbm.at[pl.ds(base + c * CH, CH)], ssem[b]
            )
        return carry

    lax.fori_loop(0, NGROUP, body, 0)
    for b in range(NBUF):
        c = NBUF * (NGROUP - 1) + b
        pltpu.make_async_copy(
            rows[b], out_hbm.at[pl.ds(base + c * CH, CH)], ssem[b]
        ).wait()


def kernel(prefix, table):
    idx = prefix.astype(jnp.int32).reshape(NW, NCHUNK, CH)
    out = _gather_kernel(idx, table)
    return out.reshape(BATCH, PREFIX_LENGTH, EMBED_DIM)


# speedup vs baseline: 2.1947x; 1.2692x over previous
import functools

import jax
import jax.numpy as jnp
from jax import lax
from jax.experimental import pallas as pl
from jax.experimental.pallas import tpu as pltpu
from jax.experimental.pallas import tpu_sc as plsc

PREFIX_LENGTH = 128
NUM_LAYERS = 24
HIDDEN_SIZE = 1024
BATCH = 16
EMBED_DIM = NUM_LAYERS * HIDDEN_SIZE
B = BATCH * PREFIX_LENGTH
V = PREFIX_LENGTH

NC, NS = 2, 16
NPHASE = 2
Q = EMBED_DIM // (NC * NPHASE)
RPT = B // NS
VPT = V // NS
CH = 1
NCHUNK = RPT // CH
NBUF = 2
NGROUP = NCHUNK // NBUF

_mesh = plsc.VectorSubcoreMesh(core_axis_name="c", subcore_axis_name="s")


@functools.partial(
    pl.kernel,
    mesh=_mesh,
    out_type=jax.ShapeDtypeStruct((B, EMBED_DIM), jnp.float32),
    scratch_types=[
        pltpu.VMEM((RPT,), jnp.int32),
        pltpu.VMEM((CH, Q), jnp.float32),
        pltpu.VMEM((CH, Q), jnp.float32),
        pltpu.VMEM_SHARED((V, Q), jnp.float32),
        pltpu.SemaphoreType.DMA,
        pltpu.SemaphoreType.DMA,
        pltpu.SemaphoreType.DMA,
        pltpu.SemaphoreType.DMA,
    ],
)
def _gather_kernel(idx_hbm, table_hbm, out_hbm, idx_v,
                   rows0, rows1, shared_tab, gsem0, gsem1, ssem0, ssem1):
    c = lax.axis_index("c")
    s = lax.axis_index("s")
    rows = (rows0, rows1)
    gsem = (gsem0, gsem1)
    ssem = (ssem0, ssem1)

    pltpu.sync_copy(idx_hbm.at[s], idx_v)
    row_base = s * RPT

    for p in range(NPHASE):
        col0 = c * (NPHASE * Q) + p * Q

        if p > 0:
            plsc.subcore_barrier()
        pltpu.sync_copy(
            table_hbm.at[pl.ds(VPT * s, VPT), pl.ds(col0, Q)],
            shared_tab.at[pl.ds(VPT * s, VPT)],
        )
        plsc.subcore_barrier()

        def body(g, carry):
            vecs = idx_v[pl.ds(g * 16, 16)]
            for lane in range(16):
                b = lane % NBUF
                k = g * 16 + lane

                def _wait_prev_store():
                    pltpu.make_async_copy(
                        rows[b],
                        out_hbm.at[pl.ds(row_base + (k - NBUF) * CH, CH),
                                   pl.ds(col0, Q)],
                        ssem[b],
                    ).wait()

                if lane >= NBUF:
                    _wait_prev_store()
                else:
                    pl.when(g > 0)(_wait_prev_store)

                v = vecs[lane]
                pltpu.async_copy(
                    shared_tab.at[pl.ds(v, CH)], rows[b], gsem[b]
                ).wait()
                pltpu.async_copy(
                    rows[b],
                    out_hbm.at[pl.ds(row_base + k * CH, CH), pl.ds(col0, Q)],
                    ssem[b],
                )
            return carry

        lax.fori_loop(0, NCHUNK // 16, body, 0)
        for b in range(NBUF):
            k = NCHUNK - NBUF + b
            pltpu.make_async_copy(
                rows[b],
                out_hbm.at[pl.ds(row_base + k * CH, CH), pl.ds(col0, Q)],
                ssem[b],
            ).wait()


def kernel(prefix, table):
    idx = prefix.astype(jnp.int32).reshape(NS, RPT)
    out = _gather_kernel(idx, table)
    return out.reshape(BATCH, PREFIX_LENGTH, EMBED_DIM)


# speedup vs baseline: 2.5705x; 1.1713x over previous
import functools

import jax
import jax.numpy as jnp
from jax import lax
from jax.experimental import pallas as pl
from jax.experimental.pallas import tpu as pltpu
from jax.experimental.pallas import tpu_sc as plsc

PREFIX_LENGTH = 128
NUM_LAYERS = 24
HIDDEN_SIZE = 1024
BATCH = 16
EMBED_DIM = NUM_LAYERS * HIDDEN_SIZE
B = BATCH * PREFIX_LENGTH
V = PREFIX_LENGTH

NC, NS = 2, 16
NPHASE = 2
Q = EMBED_DIM // (NC * NPHASE)
RPT = B // NS
VPT = V // NS
NBUF = 8
DIST = 4
NVEC = RPT // 16

_mesh = plsc.VectorSubcoreMesh(core_axis_name="c", subcore_axis_name="s")


@functools.partial(
    pl.kernel,
    mesh=_mesh,
    out_type=jax.ShapeDtypeStruct((B, EMBED_DIM), jnp.float32),
    scratch_types=(
        [pltpu.VMEM((RPT,), jnp.int32)]
        + [pltpu.VMEM((1, Q), jnp.float32) for _ in range(NBUF)]
        + [pltpu.VMEM_SHARED((V, Q), jnp.float32)]
        + [pltpu.SemaphoreType.DMA for _ in range(2 * NBUF)]
    ),
)
def _gather_kernel(idx_hbm, table_hbm, out_hbm, idx_v, *rest):
    bufs = rest[:NBUF]
    shared_tab = rest[NBUF]
    gsem = rest[NBUF + 1:NBUF + 1 + NBUF]
    ssem = rest[NBUF + 1 + NBUF:]
    c = lax.axis_index("c")
    s = lax.axis_index("s")

    pltpu.sync_copy(idx_hbm.at[s], idx_v)
    row_base = s * RPT

    def gather(v, b):
        pltpu.async_copy(shared_tab.at[pl.ds(v, 1)], bufs[b], gsem[b])

    for p in range(NPHASE):
        col0 = c * (NPHASE * Q) + p * Q
        out_at = lambda k: out_hbm.at[pl.ds(row_base + k, 1), pl.ds(col0, Q)]

        if p > 0:
            plsc.subcore_barrier()
        pltpu.sync_copy(
            table_hbm.at[pl.ds(VPT * s, VPT), pl.ds(col0, Q)],
            shared_tab.at[pl.ds(VPT * s, VPT)],
        )
        plsc.subcore_barrier()

        vec0 = idx_v[pl.ds(0, 16)]
        for k in range(DIST):
            gather(vec0[k], k % NBUF)

        def body(j, carry):
            vecs = idx_v[pl.ds(j * 16, 16)]

            def _wait_store(b2, koff):
                pltpu.make_async_copy(
                    bufs[b2], out_at(j * 16 + koff), ssem[b2]
                ).wait()

            for lane in range(16):
                b = lane % NBUF
                k = j * 16 + lane

                pltpu.make_async_copy(
                    shared_tab.at[pl.ds(0, 1)], bufs[b], gsem[b]
                ).wait()
                pltpu.async_copy(bufs[b], out_at(k), ssem[b])

                if lane + DIST < 16:
                    b2 = (lane + DIST) % NBUF
                    if lane + DIST >= NBUF:
                        _wait_store(b2, lane + DIST - NBUF)
                    else:
                        pl.when(j > 0)(
                            functools.partial(
                                _wait_store, b2, lane + DIST - NBUF))
                    gather(vecs[lane + DIST], b2)
                elif lane == 16 - DIST:
                    @pl.when(j < NVEC - 1)
                    def _prefetch_next_group():
                        vecs2 = idx_v[pl.ds(j * 16 + 16, 16)]
                        for l2 in range(16 - DIST, 16):
                            b2 = (l2 + DIST) % NBUF
                            _wait_store(b2, l2 + DIST - NBUF)
                            gather(vecs2[l2 - (16 - DIST)], b2)
            return carry

        lax.fori_loop(0, NVEC, body, 0)
        for b in range(NBUF):
            pltpu.make_async_copy(
                bufs[b], out_at(RPT - NBUF + b), ssem[b]
            ).wait()


def kernel(prefix, table):
    idx = prefix.astype(jnp.int32).reshape(NS, RPT)
    out = _gather_kernel(idx, table)
    return out.reshape(BATCH, PREFIX_LENGTH, EMBED_DIM)


# speedup vs baseline: 2.5725x; 1.0008x over previous
import functools

import jax
import jax.numpy as jnp
from jax import lax
from jax.experimental import pallas as pl
from jax.experimental.pallas import tpu as pltpu
from jax.experimental.pallas import tpu_sc as plsc

PREFIX_LENGTH = 128
NUM_LAYERS = 24
HIDDEN_SIZE = 1024
BATCH = 16
EMBED_DIM = NUM_LAYERS * HIDDEN_SIZE
B = BATCH * PREFIX_LENGTH
V = PREFIX_LENGTH

NC, NS = 2, 16
NPHASE = 2
Q = EMBED_DIM // (NC * NPHASE)
RPT = B // NS
VPT = V // NS
NBUF = 8
DIST = 6
NVEC = RPT // 16

_mesh = plsc.VectorSubcoreMesh(core_axis_name="c", subcore_axis_name="s")


@functools.partial(
    pl.kernel,
    mesh=_mesh,
    out_type=jax.ShapeDtypeStruct((B, EMBED_DIM), jnp.float32),
    scratch_types=(
        [pltpu.VMEM((RPT,), jnp.int32)]
        + [pltpu.VMEM((1, Q), jnp.float32) for _ in range(NBUF)]
        + [pltpu.VMEM_SHARED((V, Q), jnp.float32)]
        + [pltpu.SemaphoreType.DMA for _ in range(2 * NBUF)]
    ),
)
def _gather_kernel(idx_hbm, table_hbm, out_hbm, idx_v, *rest):
    bufs = rest[:NBUF]
    shared_tab = rest[NBUF]
    gsem = rest[NBUF + 1:NBUF + 1 + NBUF]
    ssem = rest[NBUF + 1 + NBUF:]
    c = lax.axis_index("c")
    s = lax.axis_index("s")

    pltpu.sync_copy(idx_hbm.at[s], idx_v)
    row_base = s * RPT

    def gather(v, b):
        pltpu.async_copy(shared_tab.at[pl.ds(v, 1)], bufs[b], gsem[b])

    for p in range(NPHASE):
        col0 = c * (NPHASE * Q) + p * Q
        out_at = lambda k: out_hbm.at[pl.ds(row_base + k, 1), pl.ds(col0, Q)]

        if p > 0:
            plsc.subcore_barrier()
        pltpu.sync_copy(
            table_hbm.at[pl.ds(VPT * s, VPT), pl.ds(col0, Q)],
            shared_tab.at[pl.ds(VPT * s, VPT)],
        )
        plsc.subcore_barrier()

        vec0 = idx_v[pl.ds(0, 16)]
        for k in range(DIST):
            gather(vec0[k], k % NBUF)

        def body(j, carry):
            vecs = idx_v[pl.ds(j * 16, 16)]
            vecs2 = idx_v[pl.ds(lax.min(j * 16 + 16, RPT - 16), 16)]

            def _wait_store(b2, koff):
                pltpu.make_async_copy(
                    bufs[b2], out_at(j * 16 + koff), ssem[b2]
                ).wait()

            for lane in range(16):
                b = lane % NBUF
                k = j * 16 + lane

                pltpu.make_async_copy(
                    shared_tab.at[pl.ds(0, 1)], bufs[b], gsem[b]
                ).wait()
                pltpu.async_copy(bufs[b], out_at(k), ssem[b])

                b2 = (lane + DIST) % NBUF
                if lane + DIST < 16:
                    if lane + DIST >= NBUF:
                        _wait_store(b2, lane + DIST - NBUF)
                    else:
                        pl.when(j > 0)(
                            functools.partial(
                                _wait_store, b2, lane + DIST - NBUF))
                    gather(vecs[lane + DIST], b2)
                else:
                    @pl.when(j < NVEC - 1)
                    def _prefetch_next_group(lane=lane, b2=b2):
                        _wait_store(b2, lane + DIST - NBUF)
                        gather(vecs2[lane + DIST - 16], b2)
            return carry

        lax.fori_loop(0, NVEC, body, 0)
        for b in range(NBUF):
            pltpu.make_async_copy(
                bufs[b], out_at(RPT - NBUF + b), ssem[b]
            ).wait()


def kernel(prefix, table):
    idx = prefix.astype(jnp.int32).reshape(NS, RPT)
    out = _gather_kernel(idx, table)
    return out.reshape(BATCH, PREFIX_LENGTH, EMBED_DIM)
